# baseline (speedup 1.0000x reference)
import functools

import jax
import jax.numpy as jnp
from jax import lax
from jax.experimental import pallas as pl
from jax.experimental.pallas import tpu as pltpu
from jax.experimental.pallas import tpu_sc as plsc

BATCH = 16384
DIM = 64
SUB = 8
NUM_CORES = 2
NUM_SUBCORES = 16
NUM_WORKERS = NUM_CORES * NUM_SUBCORES
PER_WORKER = BATCH // NUM_WORKERS
LANES = 16
NUM_GROUPS = PER_WORKER // LANES
NSETS = 8
REL_ROWS = 500


def _vec_sqrt(x):
    i = plsc.bitcast(x, jnp.int32)
    i = jnp.int32(0x5F3759DF) - lax.shift_right_logical(i, 1)
    y = plsc.bitcast(i, jnp.float32)
    half = x * jnp.float32(0.5)
    for _ in range(3):
        y = y * (jnp.float32(1.5) - half * y * y)
    return x * y


def _transe_body(heads_hbm, rels_hbm, tails_hbm, ent_hbm, relw_hbm, out_hbm,
                 hraw_v, rraw_v, traw_v, rel_v,
                 hbufs, tbufs, out_v, sems):
    wid = lax.axis_index("s") * NUM_CORES + lax.axis_index("c")
    base_w = wid * PER_WORKER
    pltpu.sync_copy(relw_hbm, rel_v)
    pltpu.sync_copy(heads_hbm.at[pl.ds(base_w, PER_WORKER)], hraw_v)
    pltpu.sync_copy(rels_hbm.at[pl.ds(base_w, PER_WORKER)], rraw_v)
    pltpu.sync_copy(tails_hbm.at[pl.ds(base_w, PER_WORKER)], traw_v)

    lane_iota = lax.iota(jnp.int32, LANES)
    one = jnp.full((LANES,), 1, jnp.int32)
    row_hi = lax.shift_right_logical(lane_iota, 3)
    row_lo = lax.bitwise_and(lane_iota, jnp.full((LANES,), SUB - 1, jnp.int32))

    def issue(g, s):
        sl = pl.ds(g * LANES, LANES)
        hraw = hraw_v[sl]
        traw = traw_v[sl]
        for j in range(LANES):
            hr = hraw[j]
            tr = traw[j]
            pltpu.async_copy(
                ent_hbm.at[lax.shift_right_logical(hr, 3),
                           lax.bitwise_and(hr, SUB - 1), :],
                hbufs[s].at[j // SUB, j % SUB], sems[s])
            pltpu.async_copy(
                ent_hbm.at[lax.shift_right_logical(tr, 3),
                           lax.bitwise_and(tr, SUB - 1), :],
                tbufs[s].at[j // SUB, j % SUB], sems[s])

    def drain(s):
        pltpu.make_async_copy(ent_hbm.at[pl.ds(0, LANES // SUB)],
                              hbufs[s], sems[s]).wait()
        pltpu.make_async_copy(ent_hbm.at[pl.ds(0, LANES // SUB)],
                              tbufs[s], sems[s]).wait()

    def compute(g, s):
        sl = pl.ds(g * LANES, LANES)
        rraw = rraw_v[sl]
        rrow = lax.shift_right_logical(rraw, 1)
        rcol = lax.bitwise_and(rraw, one) * DIM

        def dim_step(d, acc, rrow=rrow, rcol=rcol, s=s):
            hv = plsc.load_gather(hbufs[s], [row_hi, row_lo, row_hi * 0 + d])
            tv = plsc.load_gather(tbufs[s], [row_hi, row_lo, row_hi * 0 + d])
            rv = plsc.load_gather(rel_v, [rrow, rcol + d])
            diff = (hv - tv) + rv
            return acc + diff * diff

        acc = lax.fori_loop(0, DIM, dim_step, jnp.zeros((LANES,), jnp.float32))
        out_v[sl] = _vec_sqrt(acc)

    for s in range(NSETS - 1):
        issue(s, s)

    def step(k, _):
        for u in range(NSETS):
            g = k * NSETS + u
            drain(u)

            @pl.when(g + NSETS - 1 < NUM_GROUPS)
            def _(g=g, u=u):
                issue(g + NSETS - 1, (u + NSETS - 1) % NSETS)

            compute(g, u)
        return 0

    lax.fori_loop(0, NUM_GROUPS // NSETS, step, 0)
    pltpu.sync_copy(out_v, out_hbm.at[pl.ds(base_w, PER_WORKER)])


@jax.jit
def _transe(heads, relations, tails, entity_weight, relation_weight):
    ent3 = entity_weight.reshape(entity_weight.shape[0] // SUB, SUB, DIM)
    rel2 = relation_weight.reshape(relation_weight.shape[0] // 2, 2 * DIM)
    mesh = plsc.VectorSubcoreMesh(core_axis_name="c", subcore_axis_name="s")
    f = functools.partial(
        pl.kernel,
        out_type=jax.ShapeDtypeStruct((BATCH,), jnp.float32),
        mesh=mesh,
        scratch_types=[
            pltpu.VMEM((PER_WORKER,), jnp.int32),
            pltpu.VMEM((PER_WORKER,), jnp.int32),
            pltpu.VMEM((PER_WORKER,), jnp.int32),
            pltpu.VMEM((REL_ROWS, 2 * DIM), jnp.float32),
            [pltpu.VMEM((LANES // SUB, SUB, DIM), jnp.float32)
             for _ in range(NSETS)],
            [pltpu.VMEM((LANES // SUB, SUB, DIM), jnp.float32)
             for _ in range(NSETS)],
            pltpu.VMEM((PER_WORKER,), jnp.float32),
            [pltpu.SemaphoreType.DMA for _ in range(NSETS)],
        ],
        compiler_params=pltpu.CompilerParams(
            needs_layout_passes=False, use_tc_tiling_on_sc=True
        ),
    )(_transe_body)
    return f(heads, relations, tails, ent3, rel2)


def kernel(heads, relations, tails, entity_weight, relation_weight):
    return _transe(heads, relations, tails, entity_weight, relation_weight)


# speedup vs baseline: 1.0101x; 1.0101x over previous
import functools

import jax
import jax.numpy as jnp
from jax import lax
from jax.experimental import pallas as pl
from jax.experimental.pallas import tpu as pltpu
from jax.experimental.pallas import tpu_sc as plsc

BATCH = 16384
DIM = 64
SUB = 8
NUM_CORES = 2
NUM_SUBCORES = 16
NUM_WORKERS = NUM_CORES * NUM_SUBCORES
PER_WORKER = BATCH // NUM_WORKERS
LANES = 16
NUM_GROUPS = PER_WORKER // LANES
NSETS = 4
REL_ROWS = 500


def _vec_sqrt(x):
    i = plsc.bitcast(x, jnp.int32)
    i = jnp.int32(0x5F3759DF) - lax.shift_right_logical(i, 1)
    y = plsc.bitcast(i, jnp.float32)
    half = x * jnp.float32(0.5)
    for _ in range(3):
        y = y * (jnp.float32(1.5) - half * y * y)
    return x * y


def _transe_body(heads_hbm, rels_hbm, tails_hbm, ent_hbm, relw_hbm, out_hbm,
                 hraw_v, rraw_v, traw_v, rel_v,
                 hbufs, tbufs, out_v, sems):
    wid = lax.axis_index("s") * NUM_CORES + lax.axis_index("c")
    base_w = wid * PER_WORKER
    pltpu.sync_copy(relw_hbm, rel_v)
    pltpu.sync_copy(heads_hbm.at[pl.ds(base_w, PER_WORKER)], hraw_v)
    pltpu.sync_copy(rels_hbm.at[pl.ds(base_w, PER_WORKER)], rraw_v)
    pltpu.sync_copy(tails_hbm.at[pl.ds(base_w, PER_WORKER)], traw_v)

    lane_iota = lax.iota(jnp.int32, LANES)
    one = jnp.full((LANES,), 1, jnp.int32)
    row_hi = lax.shift_right_logical(lane_iota, 3)
    row_lo = lax.bitwise_and(lane_iota, jnp.full((LANES,), SUB - 1, jnp.int32))

    def issue(g, s):
        sl = pl.ds(g * LANES, LANES)
        hraw = hraw_v[sl]
        traw = traw_v[sl]
        for j in range(LANES):
            hr = hraw[j]
            tr = traw[j]
            pltpu.async_copy(
                ent_hbm.at[lax.shift_right_logical(hr, 3),
                           lax.bitwise_and(hr, SUB - 1), :],
                hbufs[s].at[j // SUB, j % SUB], sems[s])
            pltpu.async_copy(
                ent_hbm.at[lax.shift_right_logical(tr, 3),
                           lax.bitwise_and(tr, SUB - 1), :],
                tbufs[s].at[j // SUB, j % SUB], sems[s])

    def drain(s):
        pltpu.make_async_copy(ent_hbm.at[pl.ds(0, LANES // SUB)],
                              hbufs[s], sems[s]).wait()
        pltpu.make_async_copy(ent_hbm.at[pl.ds(0, LANES // SUB)],
                              tbufs[s], sems[s]).wait()

    def compute(g, s):
        sl = pl.ds(g * LANES, LANES)
        rraw = rraw_v[sl]
        rrow = lax.shift_right_logical(rraw, 1)
        rcol = lax.bitwise_and(rraw, one) * DIM

        def dim_step(d, acc, rrow=rrow, rcol=rcol, s=s):
            hv = plsc.load_gather(hbufs[s], [row_hi, row_lo, row_hi * 0 + d])
            tv = plsc.load_gather(tbufs[s], [row_hi, row_lo, row_hi * 0 + d])
            rv = plsc.load_gather(rel_v, [rrow, rcol + d])
            diff = (hv - tv) + rv
            return acc + diff * diff

        acc = lax.fori_loop(0, DIM, dim_step, jnp.zeros((LANES,), jnp.float32))
        out_v[sl] = _vec_sqrt(acc)

    for s in range(NSETS - 1):
        issue(s, s)

    def step(k, _):
        for u in range(NSETS):
            g = k * NSETS + u
            drain(u)

            @pl.when(g + NSETS - 1 < NUM_GROUPS)
            def _(g=g, u=u):
                issue(g + NSETS - 1, (u + NSETS - 1) % NSETS)

            compute(g, u)
        return 0

    lax.fori_loop(0, NUM_GROUPS // NSETS, step, 0)
    pltpu.sync_copy(out_v, out_hbm.at[pl.ds(base_w, PER_WORKER)])


@jax.jit
def _transe(heads, relations, tails, entity_weight, relation_weight):
    ent3 = entity_weight.reshape(entity_weight.shape[0] // SUB, SUB, DIM)
    rel2 = relation_weight.reshape(relation_weight.shape[0] // 2, 2 * DIM)
    mesh = plsc.VectorSubcoreMesh(core_axis_name="c", subcore_axis_name="s")
    f = functools.partial(
        pl.kernel,
        out_type=jax.ShapeDtypeStruct((BATCH,), jnp.float32),
        mesh=mesh,
        scratch_types=[
            pltpu.VMEM((PER_WORKER,), jnp.int32),
            pltpu.VMEM((PER_WORKER,), jnp.int32),
            pltpu.VMEM((PER_WORKER,), jnp.int32),
            pltpu.VMEM((REL_ROWS, 2 * DIM), jnp.float32),
            [pltpu.VMEM((LANES // SUB, SUB, DIM), jnp.float32)
             for _ in range(NSETS)],
            [pltpu.VMEM((LANES // SUB, SUB, DIM), jnp.float32)
             for _ in range(NSETS)],
            pltpu.VMEM((PER_WORKER,), jnp.float32),
            [pltpu.SemaphoreType.DMA for _ in range(NSETS)],
        ],
        compiler_params=pltpu.CompilerParams(
            needs_layout_passes=False, use_tc_tiling_on_sc=True
        ),
    )(_transe_body)
    return f(heads, relations, tails, ent3, rel2)


def kernel(heads, relations, tails, entity_weight, relation_weight):
    return _transe(heads, relations, tails, entity_weight, relation_weight)


# speedup vs baseline: 1.0159x; 1.0058x over previous
import functools

import jax
import jax.numpy as jnp
from jax import lax
from jax.experimental import pallas as pl
from jax.experimental.pallas import tpu as pltpu
from jax.experimental.pallas import tpu_sc as plsc

BATCH = 16384
DIM = 64
SUB = 8
NUM_CORES = 2
NUM_SUBCORES = 16
NUM_WORKERS = NUM_CORES * NUM_SUBCORES
PER_WORKER = BATCH // NUM_WORKERS
LANES = 16
NUM_GROUPS = PER_WORKER // LANES
NSETS = 4
REL_ROWS = 500


def _vec_sqrt(x):
    i = plsc.bitcast(x, jnp.int32)
    i = jnp.int32(0x5F3759DF) - lax.shift_right_logical(i, 1)
    y = plsc.bitcast(i, jnp.float32)
    half = x * jnp.float32(0.5)
    for _ in range(3):
        y = y * (jnp.float32(1.5) - half * y * y)
    return x * y


def _transe_body(heads_hbm, rels_hbm, tails_hbm, ent_hbm, relw_hbm, out_hbm,
                 hraw_v, rraw_v, traw_v, rel_v,
                 hbufs, tbufs, out_v, sems, rel_sem):
    wid = lax.axis_index("s") * NUM_CORES + lax.axis_index("c")
    base_w = wid * PER_WORKER
    rel_copy = pltpu.async_copy(relw_hbm, rel_v, rel_sem)
    pltpu.sync_copy(heads_hbm.at[pl.ds(base_w, PER_WORKER)], hraw_v)
    pltpu.sync_copy(rels_hbm.at[pl.ds(base_w, PER_WORKER)], rraw_v)
    pltpu.sync_copy(tails_hbm.at[pl.ds(base_w, PER_WORKER)], traw_v)

    lane_iota = lax.iota(jnp.int32, LANES)
    one = jnp.full((LANES,), 1, jnp.int32)
    row_hi = lax.shift_right_logical(lane_iota, 3)
    row_lo = lax.bitwise_and(lane_iota, jnp.full((LANES,), SUB - 1, jnp.int32))

    def issue(g, s):
        sl = pl.ds(g * LANES, LANES)
        hraw = hraw_v[sl]
        traw = traw_v[sl]
        for j in range(LANES):
            hr = hraw[j]
            tr = traw[j]
            pltpu.async_copy(
                ent_hbm.at[lax.shift_right_logical(hr, 3),
                           lax.bitwise_and(hr, SUB - 1), :],
                hbufs[s].at[j // SUB, j % SUB], sems[s])
            pltpu.async_copy(
                ent_hbm.at[lax.shift_right_logical(tr, 3),
                           lax.bitwise_and(tr, SUB - 1), :],
                tbufs[s].at[j // SUB, j % SUB], sems[s])

    def drain(s):
        pltpu.make_async_copy(ent_hbm.at[pl.ds(0, LANES // SUB)],
                              hbufs[s], sems[s]).wait()
        pltpu.make_async_copy(ent_hbm.at[pl.ds(0, LANES // SUB)],
                              tbufs[s], sems[s]).wait()

    def compute(g, s):
        sl = pl.ds(g * LANES, LANES)
        rraw = rraw_v[sl]
        rrow = lax.shift_right_logical(rraw, 1)
        rcol = lax.bitwise_and(rraw, one) * DIM

        def dim_step(d, acc, rrow=rrow, rcol=rcol, s=s):
            hv = plsc.load_gather(hbufs[s], [row_hi, row_lo, row_hi * 0 + d])
            tv = plsc.load_gather(tbufs[s], [row_hi, row_lo, row_hi * 0 + d])
            rv = plsc.load_gather(rel_v, [rrow, rcol + d])
            diff = (hv - tv) + rv
            return acc + diff * diff

        acc = lax.fori_loop(0, DIM, dim_step, jnp.zeros((LANES,), jnp.float32))
        out_v[sl] = _vec_sqrt(acc)

    for s in range(NSETS - 1):
        issue(s, s)
    rel_copy.wait()

    def step(k, _):
        for u in range(NSETS):
            g = k * NSETS + u
            drain(u)

            @pl.when(g + NSETS - 1 < NUM_GROUPS)
            def _(g=g, u=u):
                issue(g + NSETS - 1, (u + NSETS - 1) % NSETS)

            compute(g, u)
        return 0

    lax.fori_loop(0, NUM_GROUPS // NSETS, step, 0)
    pltpu.sync_copy(out_v, out_hbm.at[pl.ds(base_w, PER_WORKER)])


@jax.jit
def _transe(heads, relations, tails, entity_weight, relation_weight):
    ent3 = entity_weight.reshape(entity_weight.shape[0] // SUB, SUB, DIM)
    rel2 = relation_weight.reshape(relation_weight.shape[0] // 2, 2 * DIM)
    mesh = plsc.VectorSubcoreMesh(core_axis_name="c", subcore_axis_name="s")
    f = functools.partial(
        pl.kernel,
        out_type=jax.ShapeDtypeStruct((BATCH,), jnp.float32),
        mesh=mesh,
        scratch_types=[
            pltpu.VMEM((PER_WORKER,), jnp.int32),
            pltpu.VMEM((PER_WORKER,), jnp.int32),
            pltpu.VMEM((PER_WORKER,), jnp.int32),
            pltpu.VMEM((REL_ROWS, 2 * DIM), jnp.float32),
            [pltpu.VMEM((LANES // SUB, SUB, DIM), jnp.float32)
             for _ in range(NSETS)],
            [pltpu.VMEM((LANES // SUB, SUB, DIM), jnp.float32)
             for _ in range(NSETS)],
            pltpu.VMEM((PER_WORKER,), jnp.float32),
            [pltpu.SemaphoreType.DMA for _ in range(NSETS)],
            pltpu.SemaphoreType.DMA,
        ],
        compiler_params=pltpu.CompilerParams(
            needs_layout_passes=False, use_tc_tiling_on_sc=True
        ),
    )(_transe_body)
    return f(heads, relations, tails, ent3, rel2)


def kernel(heads, relations, tails, entity_weight, relation_weight):
    return _transe(heads, relations, tails, entity_weight, relation_weight)
